# baseline (speedup 1.0000x reference)
import functools

import jax
import jax.numpy as jnp
from jax import lax
from jax.experimental import pallas as pl
from jax.experimental.pallas import tpu as pltpu
from jax.experimental.pallas import tpu_sc as plsc

_BETA = 0.3
_C = 16
_N, _D = 4096, 512
_NC, _NS, _L = 2, 16, 16
_NW = _NC * _NS
_RPW = _N // _NW
_JC = _D // _L

_f32 = jnp.float32
_i32 = jnp.int32


_NB = 4
_RPB = _RPW // _NB
_HC = 16


def _sc1_body(e_hbm, lab_hbm, s_out,
              e_v, lab_v, s_loc, cur_s, beg_s, perm_s,
              row16_v, acc_v, s_sh, sems):
    c = lax.axis_index("c")
    s = lax.axis_index("s")
    wid = s * _NC + c
    base = wid * _RPW
    pltpu.sync_copy(lab_hbm.at[pl.ds(base, _RPW)], lab_v)
    cps = [pltpu.async_copy(
        e_hbm.at[pl.ds(base + b * _RPB, _RPB)],
        e_v.at[pl.ds(b * _RPB, _RPB)], sems[b]) for b in range(_NB)]

    zero = jnp.zeros((_L,), _f32)

    for cc in range(_C):
        cur_s[cc] = 0
    for g in range(_RPW // _L):
        lab_grp = lab_v[pl.ds(g * _L, _L)]
        for p in range(_L):
            l = lab_grp[p]
            cur_s[l] = cur_s[l] + 1
    off = 0
    for cc in range(_C):
        beg_s[cc] = off
        off = off + cur_s[cc]
        cur_s[cc] = beg_s[cc]
    beg_s[_C] = _RPW
    for g in range(_RPW // _L):
        lab_grp = lab_v[pl.ds(g * _L, _L)]
        for p in range(_L):
            l = lab_grp[p]
            pos = cur_s[l]
            cur_s[l] = pos + 1
            perm_s[pos] = g * _L + p

    for b in range(_NB):
        cps[b].wait()

    for half in range(_D // (_HC * _L)):
        for cc in range(_C):
            lo = beg_s[cc]
            hi = beg_s[cc + 1]

            def _body(k, carry, _half=half):
                r = perm_s[k]
                return tuple(
                    carry[j] + e_v[r, pl.ds((_half * _HC + j) * _L, _L)]
                    for j in range(_HC))

            accs = lax.fori_loop(lo, hi, _body, (zero,) * _HC)
            for j in range(_HC):
                s_loc[cc, pl.ds((half * _HC + j) * _L, _L)] = accs[j]

    pltpu.sync_copy(s_loc, s_sh.at[s])
    plsc.subcore_barrier()

    cps2 = [pltpu.async_copy(s_sh.at[t, s], row16_v.at[t], sems[t % _NB])
            for t in range(_NS)]
    for cp in cps2:
        cp.wait()
    for j in range(_JC):
        v = row16_v[0, pl.ds(j * _L, _L)]
        for t in range(1, _NS):
            v = v + row16_v[t, pl.ds(j * _L, _L)]
        acc_v[pl.ds(j * _L, _L)] = v
    pltpu.sync_copy(acc_v, s_out.at[c, s])


_sc1 = functools.partial(
    pl.kernel,
    out_type=jax.ShapeDtypeStruct((_NC, _C, _D), _f32),
    mesh=plsc.VectorSubcoreMesh(core_axis_name="c", subcore_axis_name="s"),
    scratch_types=(
        pltpu.VMEM((_RPW, _D), _f32),
        pltpu.VMEM((_RPW,), _i32),
        pltpu.VMEM((_C, _D), _f32),
        pltpu.SMEM((_C,), _i32),
        pltpu.SMEM((_C + 1,), _i32),
        pltpu.SMEM((_RPW,), _i32),
        pltpu.VMEM((_NS, _D), _f32),
        pltpu.VMEM((_D,), _f32),
        pltpu.VMEM_SHARED((_NS, _C, _D), _f32),
        [pltpu.SemaphoreType.DMA] * _NB,
    ),
)(_sc1_body)


def _tc_body(e_ref, lab_ref, s_ref, out_ref):
    E = e_ref[...]
    lab = lab_ref[...]
    S = jnp.sum(s_ref[...], axis=0)

    classes = lax.broadcasted_iota(_i32, (1, _C), 1)
    M = (lab == classes).astype(_f32)
    cnt = jnp.sum(M, axis=0, keepdims=True).reshape(_C, 1)
    sq = jnp.sum(E * E, axis=1, keepdims=True)
    q = lax.dot_general(M, sq, (((0,), (0,)), ((), ())),
                        preferred_element_type=_f32)

    safe = jnp.maximum(cnt, 1.0)
    mu = S / safe
    n2 = jnp.sum(mu * mu, axis=1, keepdims=True)
    mean_d = q / safe - n2
    coeff = n2 - mean_d

    D = lax.dot_general(E, mu, (((1,), (1,)), ((), ())),
                        preferred_element_type=_f32)
    dot_i = jnp.sum(D * M, axis=1, keepdims=True)
    cof = lax.dot_general(M, coeff, (((1,), (0,)), ((), ())),
                          preferred_element_type=_f32)
    t = sq - 2.0 * dot_i + cof
    T2 = lax.dot_general(M, t * t, (((0,), (0,)), ((), ())),
                         preferred_element_type=_f32)

    var = T2 / jnp.maximum(cnt - 1.0, 1.0)
    total = jnp.sum(jnp.where(cnt > 1.0, var, 0.0))
    nu = jnp.sum((cnt > 0.0).astype(_f32))
    loss = _BETA * total / jnp.maximum(nu, 1.0)
    out_ref[...] = jnp.full((1, 1), loss, dtype=_f32)


def kernel(embeddings, labels):
    lab = labels.astype(_i32)
    s_part = _sc1(embeddings, lab)
    out = pl.pallas_call(
        _tc_body,
        out_shape=jax.ShapeDtypeStruct((1, 1), _f32),
    )(embeddings, lab.reshape(_N, 1), s_part)
    return out[0, 0]


# speedup vs baseline: 1.0422x; 1.0422x over previous
import functools

import jax
import jax.numpy as jnp
from jax import lax
from jax.experimental import pallas as pl
from jax.experimental.pallas import tpu as pltpu
from jax.experimental.pallas import tpu_sc as plsc

_BETA = 0.3
_C = 16
_N, _D = 4096, 512
_NC, _NS, _L = 2, 16, 16
_NW = _NC * _NS
_RPW = _N // _NW
_JC = _D // _L

_f32 = jnp.float32
_i32 = jnp.int32


_NB = 4
_RPB = _RPW // _NB
_HC = 16


def _sc1_body(e_hbm, lab_hbm, s_out,
              e_v, lab_v, s_loc, cur_s, beg_s, perm_s, sems):
    c = lax.axis_index("c")
    s = lax.axis_index("s")
    wid = s * _NC + c
    base = wid * _RPW
    pltpu.sync_copy(lab_hbm.at[pl.ds(base, _RPW)], lab_v)
    cps = [pltpu.async_copy(
        e_hbm.at[pl.ds(base + b * _RPB, _RPB)],
        e_v.at[pl.ds(b * _RPB, _RPB)], sems[b]) for b in range(_NB)]

    zero = jnp.zeros((_L,), _f32)

    for cc in range(_C):
        cur_s[cc] = 0
    for g in range(_RPW // _L):
        lab_grp = lab_v[pl.ds(g * _L, _L)]
        for p in range(_L):
            l = lab_grp[p]
            cur_s[l] = cur_s[l] + 1
    off = 0
    for cc in range(_C):
        beg_s[cc] = off
        off = off + cur_s[cc]
        cur_s[cc] = beg_s[cc]
    beg_s[_C] = _RPW
    for g in range(_RPW // _L):
        lab_grp = lab_v[pl.ds(g * _L, _L)]
        for p in range(_L):
            l = lab_grp[p]
            pos = cur_s[l]
            cur_s[l] = pos + 1
            perm_s[pos] = g * _L + p

    for b in range(_NB):
        cps[b].wait()

    for half in range(_D // (_HC * _L)):
        for cc in range(_C):
            lo = beg_s[cc]
            hi = beg_s[cc + 1]

            def _body(k, carry, _half=half):
                r = perm_s[k]
                return tuple(
                    carry[j] + e_v[r, pl.ds((_half * _HC + j) * _L, _L)]
                    for j in range(_HC))

            accs = lax.fori_loop(lo, hi, _body, (zero,) * _HC)
            for j in range(_HC):
                s_loc[cc, pl.ds((half * _HC + j) * _L, _L)] = accs[j]

    pltpu.sync_copy(s_loc, s_out.at[wid])


_sc1 = functools.partial(
    pl.kernel,
    out_type=jax.ShapeDtypeStruct((_NW, _C, _D), _f32),
    mesh=plsc.VectorSubcoreMesh(core_axis_name="c", subcore_axis_name="s"),
    scratch_types=(
        pltpu.VMEM((_RPW, _D), _f32),
        pltpu.VMEM((_RPW,), _i32),
        pltpu.VMEM((_C, _D), _f32),
        pltpu.SMEM((_C,), _i32),
        pltpu.SMEM((_C + 1,), _i32),
        pltpu.SMEM((_RPW,), _i32),
        [pltpu.SemaphoreType.DMA] * _NB,
    ),
)(_sc1_body)


def _tca_body(e_ref, lab_ref, sq_ref, cq_ref):
    E = e_ref[...]
    lab = lab_ref[...]
    classes = lax.broadcasted_iota(_i32, (1, _C), 1)
    M = (lab == classes).astype(_f32)
    sq = jnp.sum(E * E, axis=1, keepdims=True)
    sq_ref[...] = sq
    cnt = jnp.sum(M, axis=0, keepdims=True)
    q = lax.dot_general(M, sq, (((0,), (0,)), ((), ())),
                        preferred_element_type=_f32)
    cq_ref[0:1, :] = cnt
    cq_ref[1:2, :] = q.reshape(1, _C)


def _tcb_body(e_ref, lab_ref, s_ref, sq_ref, cq_ref, out_ref):
    E = e_ref[...]
    lab = lab_ref[...]
    S = jnp.sum(s_ref[...], axis=0)
    sq = sq_ref[...]
    cnt = cq_ref[0:1, :].reshape(_C, 1)
    q = cq_ref[1:2, :].reshape(_C, 1)

    safe = jnp.maximum(cnt, 1.0)
    mu = S / safe
    n2 = jnp.sum(mu * mu, axis=1, keepdims=True)
    mean_d = q / safe - n2
    coeff = n2 - mean_d

    classes = lax.broadcasted_iota(_i32, (1, _C), 1)
    M = (lab == classes).astype(_f32)
    D = lax.dot_general(E, mu, (((1,), (1,)), ((), ())),
                        preferred_element_type=_f32)
    dot_i = jnp.sum(D * M, axis=1, keepdims=True)
    cof = lax.dot_general(M, coeff, (((1,), (0,)), ((), ())),
                          preferred_element_type=_f32)
    t = sq - 2.0 * dot_i + cof
    T2 = lax.dot_general(M, t * t, (((0,), (0,)), ((), ())),
                         preferred_element_type=_f32)

    var = T2 / jnp.maximum(cnt - 1.0, 1.0)
    total = jnp.sum(jnp.where(cnt > 1.0, var, 0.0))
    nu = jnp.sum((cnt > 0.0).astype(_f32))
    loss = _BETA * total / jnp.maximum(nu, 1.0)
    out_ref[...] = jnp.full((1, 1), loss, dtype=_f32)


def kernel(embeddings, labels):
    lab = labels.astype(_i32)
    lab2d = lab.reshape(_N, 1)
    s_part = _sc1(embeddings, lab)
    sq, cq = pl.pallas_call(
        _tca_body,
        out_shape=(jax.ShapeDtypeStruct((_N, 1), _f32),
                   jax.ShapeDtypeStruct((2, _C), _f32)),
    )(embeddings, lab2d)
    out = pl.pallas_call(
        _tcb_body,
        out_shape=jax.ShapeDtypeStruct((1, 1), _f32),
    )(embeddings, lab2d, s_part, sq, cq)
    return out[0, 0]
